# baseline (speedup 1.0000x reference)
import jax
import jax.numpy as jnp
from jax import lax
from jax.experimental import pallas as pl
from jax.experimental.pallas import tpu as pltpu
from jax.experimental.pallas import tpu_sc as plsc

_B = 16384
_CF = 16
_SUB = _CF * _CF
_NV = 1000000
_NC, _NS = 2, 16
_NW = _NC * _NS
_BPW = _B // _NW
_CHUNK = 16
_NCHUNK = _BPW // _CHUNK
_LAST_BLK = _NV // 128 - 1
_TAIL_START = _NV - 128
_TAIL_CUT = (_LAST_BLK + 1) * 128
_TAIL_OFF = _CHUNK * 128


def _gather_body(u_idx, i_idx, u_tab, i_tab, u_tail, i_tail, u_out, i_out,
                 uidx_v, iidx_v, ublk, iblk, ustg, istg,
                 usem, isem, uosem, iosem):
    wid = lax.axis_index("s") * _NC + lax.axis_index("c")
    base = pl.multiple_of(wid * _BPW, 128)
    pltpu.sync_copy(u_idx.at[pl.ds(base, _BPW)], uidx_v)
    pltpu.sync_copy(i_idx.at[pl.ds(base, _BPW)], iidx_v)
    pltpu.sync_copy(u_tail, ublk.at[:, pl.ds(_TAIL_OFF, 128)])
    pltpu.sync_copy(i_tail, iblk.at[:, pl.ds(_TAIL_OFF, 128)])

    lastb = jnp.full((_CHUNK,), _LAST_BLK, jnp.int32)

    def fire(tab, idx_v, blk, sem, g):
        iv = idx_v[pl.ds(g * _CHUNK, _CHUNK)]
        bv = jnp.minimum(lax.shift_right_logical(iv, 7), lastb)
        for k in range(_CHUNK):
            pltpu.async_copy(
                tab.at[:, pl.ds(pl.multiple_of(bv[k] * 128, 128), 128)],
                blk.at[:, pl.ds(k * 128, 128)], sem)

    def drain_blocks(tab, blk, sem):
        pltpu.make_async_copy(tab.at[:, pl.ds(0, _CHUNK * 128)],
                              blk.at[:, pl.ds(0, _CHUNK * 128)], sem).wait()

    def stage(idx_v, blk, stg, g):
        iv = idx_v[pl.ds(g * _CHUNK, _CHUNK)]
        bv = jnp.minimum(lax.shift_right_logical(iv, 7), lastb)
        slotv = lax.iota(jnp.int32, _CHUNK) * 128 + (iv - bv * 128)
        tailv = _TAIL_OFF + (iv - _TAIL_START)
        absv = jnp.where(iv >= _TAIL_CUT, tailv, slotv)
        startv = lax.shift_left(lax.shift_right_logical(absv, 4), 4)
        for k in range(_CHUNK):
            start = pl.multiple_of(startv[k], 16)
            for d in range(_CF):
                stg[k, pl.ds(d * _CF, _CF)] = blk[d, pl.ds(start, 16)]

    def drain_out(stg, out, osem):
        pltpu.make_async_copy(stg, out.at[pl.ds(0, _CHUNK), :], osem).wait()

    fire(u_tab, uidx_v, ublk, usem, 0)
    fire(i_tab, iidx_v, iblk, isem, 0)

    def body(g, carry):
        row0 = pl.multiple_of(base + g * _CHUNK, 16)
        drain_blocks(u_tab, ublk, usem)

        @pl.when(g > 0)
        def _():
            drain_out(ustg, u_out, uosem)

        stage(uidx_v, ublk, ustg, g)
        pltpu.async_copy(ustg, u_out.at[pl.ds(row0, _CHUNK), :], uosem)

        @pl.when(g < _NCHUNK - 1)
        def _():
            fire(u_tab, uidx_v, ublk, usem, g + 1)

        drain_blocks(i_tab, iblk, isem)

        @pl.when(g > 0)
        def _():
            drain_out(istg, i_out, iosem)

        stage(iidx_v, iblk, istg, g)
        pltpu.async_copy(istg, i_out.at[pl.ds(row0, _CHUNK), :], iosem)

        @pl.when(g < _NCHUNK - 1)
        def _():
            fire(i_tab, iidx_v, iblk, isem, g + 1)

        return carry

    lax.fori_loop(0, _NCHUNK, body, 0)
    drain_out(ustg, u_out, uosem)
    drain_out(istg, i_out, iosem)


def _make_gather():
    return pl.kernel(
        _gather_body,
        out_type=(jax.ShapeDtypeStruct((_B, _SUB), jnp.float32),
                  jax.ShapeDtypeStruct((_B, _SUB), jnp.float32)),
        mesh=plsc.VectorSubcoreMesh(core_axis_name="c", subcore_axis_name="s",
                                    num_cores=_NC, num_subcores=_NS),
        scratch_types=[
            pltpu.VMEM((_BPW,), jnp.int32),
            pltpu.VMEM((_BPW,), jnp.int32),
            pltpu.VMEM((_CF, (_CHUNK + 1) * 128), jnp.float32),
            pltpu.VMEM((_CF, (_CHUNK + 1) * 128), jnp.float32),
            pltpu.VMEM((_CHUNK, _SUB), jnp.float32),
            pltpu.VMEM((_CHUNK, _SUB), jnp.float32),
            pltpu.SemaphoreType.DMA,
            pltpu.SemaphoreType.DMA,
            pltpu.SemaphoreType.DMA,
            pltpu.SemaphoreType.DMA,
        ],
    )


_BLK = 2048


def _dot(a, b):
    return jnp.dot(a, b, preferred_element_type=jnp.float32)


def _ce_body(feat, cw0, cb0, cw1, cb1, cw2, cb2, out):
    h = jnp.maximum(_dot(feat[...], cw0[...]) + cb0[...], 0.0)
    h = jnp.maximum(_dot(h, cw1[...]) + cb1[...], 0.0)
    out[...] = _dot(h, cw2[...]) + cb2[...]


def _mlp_body(subu, subi, c, ru, ri, selm, w0, b0, w1, b1, w2, b2, fw, fb,
              out):
    idxm = (lax.broadcasted_iota(jnp.int32, (_BLK, _CF), 1) * _CF) & 127
    dlow = lax.broadcasted_iota(jnp.int32, (_BLK, _CF), 1) < 8

    def _select(sub, r):
        idx = idxm + r[...]
        lo = jnp.take_along_axis(sub[:, :128], idx, axis=1)
        hi = jnp.take_along_axis(sub[:, 128:], idx, axis=1)
        return jnp.where(dlow, lo, hi)

    uemb = _select(subu[...], ru)
    iemb = _select(subi[...], ri)
    combined = jnp.concatenate([uemb, iemb, c[...]], axis=1)
    m = jnp.maximum(_dot(combined, w0[...]) + b0[...], 0.0)
    m = jnp.maximum(_dot(m, w1[...]) + b1[...], 0.0)
    m = jnp.maximum(_dot(m, w2[...]) + b2[...], 0.0)
    out[...] = (_dot(m, fw[...]) + fb[...])[:, 0]


def _full(shape):
    if len(shape) == 1:
        return pl.BlockSpec(shape, lambda i: (0,))
    return pl.BlockSpec(shape, lambda i: (0, 0))


def _ce(feat, cw0, cb0, cw1, cb1, cw2, cb2):
    return pl.pallas_call(
        _ce_body,
        grid=(_B // _BLK,),
        in_specs=[
            pl.BlockSpec((_BLK, 128), lambda i: (i, 0)),
            _full((128, 32)), _full((1, 32)),
            _full((32, 16)), _full((1, 16)),
            _full((16, _CF)), _full((1, _CF)),
        ],
        out_specs=pl.BlockSpec((_BLK, _CF), lambda i: (i, 0)),
        out_shape=jax.ShapeDtypeStruct((_B, _CF), jnp.float32),
    )(feat, cw0, cb0, cw1, cb1, cw2, cb2)


def _mlp(subu, subi, c, ru, ri, selm, w0, b0, w1, b1, w2, b2, fw, fb):
    return pl.pallas_call(
        _mlp_body,
        grid=(_B // _BLK,),
        in_specs=[
            pl.BlockSpec((_BLK, _SUB), lambda i: (i, 0)),
            pl.BlockSpec((_BLK, _SUB), lambda i: (i, 0)),
            pl.BlockSpec((_BLK, _CF), lambda i: (i, 0)),
            pl.BlockSpec((_BLK, 1), lambda i: (i, 0)),
            pl.BlockSpec((_BLK, 1), lambda i: (i, 0)),
            _full((_SUB, _CF)),
            _full((3 * _CF, 32)), _full((1, 32)),
            _full((32, 16)), _full((1, 16)),
            _full((16, 8)), _full((1, 8)),
            _full((8, 1)), _full((1,)),
        ],
        out_specs=pl.BlockSpec((_BLK,), lambda i: (i,)),
        out_shape=jax.ShapeDtypeStruct((_B,), jnp.float32),
    )(subu, subi, c, ru, ri, selm, w0, b0, w1, b1, w2, b2, fw, fb)


def kernel(user_indices, item_indices, item_features, user_table, item_table,
           ce_w0, ce_b0, ce_w1, ce_b1, ce_w2, ce_b2,
           mlp_w0, mlp_b0, mlp_w1, mlp_b1, mlp_w2, mlp_b2,
           fin_w, fin_b):
    u_tail = user_table[_TAIL_START:].T
    i_tail = item_table[_TAIL_START:].T
    subu, subi = _make_gather()(
        user_indices, item_indices, user_table.T, item_table.T, u_tail, i_tail)
    c = _ce(item_features, ce_w0, ce_b0.reshape(1, -1),
            ce_w1, ce_b1.reshape(1, -1), ce_w2, ce_b2.reshape(1, -1))
    ru = (user_indices & 15).astype(jnp.int32).reshape(_B, 1)
    ri = (item_indices & 15).astype(jnp.int32).reshape(_B, 1)
    selm = jnp.repeat(jnp.eye(_CF, dtype=jnp.float32), _CF, axis=0)
    return _mlp(
        subu, subi, c, ru, ri, selm, mlp_w0,
        mlp_b0.reshape(1, -1), mlp_w1, mlp_b1.reshape(1, -1),
        mlp_w2, mlp_b2.reshape(1, -1),
        fin_w, fin_b,
    )


# speedup vs baseline: 1.0650x; 1.0650x over previous
import jax
import jax.numpy as jnp
from jax import lax
from jax.experimental import pallas as pl
from jax.experimental.pallas import tpu as pltpu
from jax.experimental.pallas import tpu_sc as plsc

_B = 16384
_CF = 16
_SUB = _CF * _CF
_NV = 1000000
_NC, _NS = 2, 16
_NW = _NC * _NS
_BPW = _B // _NW
_CHUNK = 16
_NCHUNK = _BPW // _CHUNK
_LAST_BLK = _NV // 128 - 1
_TAIL_START = _NV - 128
_TAIL_CUT = (_LAST_BLK + 1) * 128
_TAIL_OFF = _CHUNK * 128


_HC = _CHUNK // 2
_HTAIL = _HC * 128


def _gather_body(u_idx, i_idx, u_tab, i_tab, u_tail, i_tail, u_out, i_out,
                 uidx_v, iidx_v, ublkA, ublkB, iblkA, iblkB,
                 ustgA, ustgB, istgA, istgB,
                 usemA, usemB, isemA, isemB,
                 uosemA, uosemB, iosemA, iosemB):
    wid = lax.axis_index("s") * _NC + lax.axis_index("c")
    base = pl.multiple_of(wid * _BPW, 128)
    pltpu.sync_copy(u_idx.at[pl.ds(base, _BPW)], uidx_v)
    pltpu.sync_copy(i_idx.at[pl.ds(base, _BPW)], iidx_v)
    for blk, tail in ((ublkA, u_tail), (ublkB, u_tail),
                      (iblkA, i_tail), (iblkB, i_tail)):
        pltpu.sync_copy(tail, blk.at[:, pl.ds(_HTAIL, 128)])

    lastb = jnp.full((_CHUNK,), _LAST_BLK, jnp.int32)

    def chunk_vecs(idx_v, g):
        iv = idx_v[pl.ds(g * _CHUNK, _CHUNK)]
        bv = jnp.minimum(lax.shift_right_logical(iv, 7), lastb)
        return iv, bv

    def fire(tab, bv, blk, sem, half):
        for k in range(_HC):
            pltpu.async_copy(
                tab.at[:, pl.ds(pl.multiple_of(bv[half * _HC + k] * 128, 128),
                                128)],
                blk.at[:, pl.ds(k * 128, 128)], sem)

    def drain_blocks(tab, blk, sem):
        pltpu.make_async_copy(tab.at[:, pl.ds(0, _HC * 128)],
                              blk.at[:, pl.ds(0, _HC * 128)], sem).wait()

    def stage(iv, bv, blk, stg, half):
        halves = lax.iota(jnp.int32, _CHUNK) & 7
        slotv = halves * 128 + (iv - bv * 128)
        tailv = _HTAIL + (iv - _TAIL_START)
        absv = jnp.where(iv >= _TAIL_CUT, tailv, slotv)
        startv = lax.shift_left(lax.shift_right_logical(absv, 4), 4)
        for k in range(_HC):
            start = pl.multiple_of(startv[half * _HC + k], 16)
            for d in range(_CF):
                stg[k, pl.ds(d * _CF, _CF)] = blk[d, pl.ds(start, 16)]

    def drain_out(stg, out, osem):
        pltpu.make_async_copy(stg, out.at[pl.ds(0, _HC), :], osem).wait()

    iv_u0, bv_u0 = chunk_vecs(uidx_v, 0)
    iv_i0, bv_i0 = chunk_vecs(iidx_v, 0)
    fire(u_tab, bv_u0, ublkA, usemA, 0)
    fire(i_tab, bv_i0, iblkA, isemA, 0)
    fire(u_tab, bv_u0, ublkB, usemB, 1)
    fire(i_tab, bv_i0, iblkB, isemB, 1)

    def body(g, carry):
        iv_u, bv_u = chunk_vecs(uidx_v, g)
        iv_i, bv_i = chunk_vecs(iidx_v, g)
        row0 = pl.multiple_of(base + g * _CHUNK, 16)

        def phase(tab, iv, bv, blk, stg, sem, osem, out, half, nbv):
            drain_blocks(tab, blk, sem)

            @pl.when(g > 0)
            def _():
                drain_out(stg, out, osem)

            stage(iv, bv, blk, stg, half)
            pltpu.async_copy(
                stg, out.at[pl.ds(row0 + half * _HC, _HC), :], osem)

            @pl.when(g < _NCHUNK - 1)
            def _():
                fire(tab, nbv, blk, sem, half)

        ng = jnp.minimum(g + 1, _NCHUNK - 1)
        _, nbv_u = chunk_vecs(uidx_v, ng)
        _, nbv_i = chunk_vecs(iidx_v, ng)

        phase(u_tab, iv_u, bv_u, ublkA, ustgA, usemA, uosemA, u_out, 0, nbv_u)
        phase(i_tab, iv_i, bv_i, iblkA, istgA, isemA, iosemA, i_out, 0, nbv_i)
        phase(u_tab, iv_u, bv_u, ublkB, ustgB, usemB, uosemB, u_out, 1, nbv_u)
        phase(i_tab, iv_i, bv_i, iblkB, istgB, isemB, iosemB, i_out, 1, nbv_i)
        return carry

    lax.fori_loop(0, _NCHUNK, body, 0)
    drain_out(ustgA, u_out, uosemA)
    drain_out(istgA, i_out, iosemA)
    drain_out(ustgB, u_out, uosemB)
    drain_out(istgB, i_out, iosemB)


def _make_gather():
    return pl.kernel(
        _gather_body,
        out_type=(jax.ShapeDtypeStruct((_B, _SUB), jnp.float32),
                  jax.ShapeDtypeStruct((_B, _SUB), jnp.float32)),
        mesh=plsc.VectorSubcoreMesh(core_axis_name="c", subcore_axis_name="s",
                                    num_cores=_NC, num_subcores=_NS),
        scratch_types=(
            [pltpu.VMEM((_BPW,), jnp.int32)] * 2
            + [pltpu.VMEM((_CF, (_HC + 1) * 128), jnp.float32)] * 4
            + [pltpu.VMEM((_HC, _SUB), jnp.float32)] * 4
            + [pltpu.SemaphoreType.DMA] * 8
        ),
    )


_BLK = 2048


def _dot(a, b):
    return jnp.dot(a, b, preferred_element_type=jnp.float32)


def _ce_body(feat, cw0, cb0, cw1, cb1, cw2, cb2, out):
    h = jnp.maximum(_dot(feat[...], cw0[...]) + cb0[...], 0.0)
    h = jnp.maximum(_dot(h, cw1[...]) + cb1[...], 0.0)
    out[...] = _dot(h, cw2[...]) + cb2[...]


def _mlp_body(subu, subi, c, ru, ri, selm, w0, b0, w1, b1, w2, b2, fw, fb,
              out):
    idxm = (lax.broadcasted_iota(jnp.int32, (_BLK, _CF), 1) * _CF) & 127
    dlow = lax.broadcasted_iota(jnp.int32, (_BLK, _CF), 1) < 8

    def _select(sub, r):
        idx = idxm + r[...]
        lo = jnp.take_along_axis(sub[:, :128], idx, axis=1)
        hi = jnp.take_along_axis(sub[:, 128:], idx, axis=1)
        return jnp.where(dlow, lo, hi)

    uemb = _select(subu[...], ru)
    iemb = _select(subi[...], ri)
    combined = jnp.concatenate([uemb, iemb, c[...]], axis=1)
    m = jnp.maximum(_dot(combined, w0[...]) + b0[...], 0.0)
    m = jnp.maximum(_dot(m, w1[...]) + b1[...], 0.0)
    m = jnp.maximum(_dot(m, w2[...]) + b2[...], 0.0)
    out[...] = (_dot(m, fw[...]) + fb[...])[:, 0]


def _full(shape):
    if len(shape) == 1:
        return pl.BlockSpec(shape, lambda i: (0,))
    return pl.BlockSpec(shape, lambda i: (0, 0))


def _ce(feat, cw0, cb0, cw1, cb1, cw2, cb2):
    return pl.pallas_call(
        _ce_body,
        grid=(_B // _BLK,),
        in_specs=[
            pl.BlockSpec((_BLK, 128), lambda i: (i, 0)),
            _full((128, 32)), _full((1, 32)),
            _full((32, 16)), _full((1, 16)),
            _full((16, _CF)), _full((1, _CF)),
        ],
        out_specs=pl.BlockSpec((_BLK, _CF), lambda i: (i, 0)),
        out_shape=jax.ShapeDtypeStruct((_B, _CF), jnp.float32),
    )(feat, cw0, cb0, cw1, cb1, cw2, cb2)


def _mlp(subu, subi, c, ru, ri, selm, w0, b0, w1, b1, w2, b2, fw, fb):
    return pl.pallas_call(
        _mlp_body,
        grid=(_B // _BLK,),
        in_specs=[
            pl.BlockSpec((_BLK, _SUB), lambda i: (i, 0)),
            pl.BlockSpec((_BLK, _SUB), lambda i: (i, 0)),
            pl.BlockSpec((_BLK, _CF), lambda i: (i, 0)),
            pl.BlockSpec((_BLK, 1), lambda i: (i, 0)),
            pl.BlockSpec((_BLK, 1), lambda i: (i, 0)),
            _full((_SUB, _CF)),
            _full((3 * _CF, 32)), _full((1, 32)),
            _full((32, 16)), _full((1, 16)),
            _full((16, 8)), _full((1, 8)),
            _full((8, 1)), _full((1,)),
        ],
        out_specs=pl.BlockSpec((_BLK,), lambda i: (i,)),
        out_shape=jax.ShapeDtypeStruct((_B,), jnp.float32),
    )(subu, subi, c, ru, ri, selm, w0, b0, w1, b1, w2, b2, fw, fb)


def kernel(user_indices, item_indices, item_features, user_table, item_table,
           ce_w0, ce_b0, ce_w1, ce_b1, ce_w2, ce_b2,
           mlp_w0, mlp_b0, mlp_w1, mlp_b1, mlp_w2, mlp_b2,
           fin_w, fin_b):
    u_tail = user_table[_TAIL_START:].T
    i_tail = item_table[_TAIL_START:].T
    subu, subi = _make_gather()(
        user_indices, item_indices, user_table.T, item_table.T, u_tail, i_tail)
    c = _ce(item_features, ce_w0, ce_b0.reshape(1, -1),
            ce_w1, ce_b1.reshape(1, -1), ce_w2, ce_b2.reshape(1, -1))
    ru = (user_indices & 15).astype(jnp.int32).reshape(_B, 1)
    ri = (item_indices & 15).astype(jnp.int32).reshape(_B, 1)
    selm = jnp.repeat(jnp.eye(_CF, dtype=jnp.float32), _CF, axis=0)
    return _mlp(
        subu, subi, c, ru, ri, selm, mlp_w0,
        mlp_b0.reshape(1, -1), mlp_w1, mlp_b1.reshape(1, -1),
        mlp_w2, mlp_b2.reshape(1, -1),
        fin_w, fin_b,
    )
